# baseline (speedup 1.0000x reference)
import functools

import jax
import jax.numpy as jnp
from jax import lax
from jax.experimental import pallas as pl
from jax.experimental.pallas import tpu as pltpu
from jax.experimental.pallas import tpu_sc as plsc

_NUM_CLASSES = 1000
_BATCH = 16384
_NC = 2
_NS = 16
_L = 16
_NW = _NC * _NS
_ROWS_PER_W = _BATCH // _NW
_TILE_ROWS = 32
_TILES = _ROWS_PER_W // _TILE_ROWS
_GROUPS = _TILE_ROWS // _L
_UNROLL = 8
_STEPS = _NUM_CLASSES // _UNROLL


def _tec_body(true_hbm, logits_hbm, out_hbm, buf, true_v, acc_v, dma_sem):
    wid = lax.axis_index("s") * _NC + lax.axis_index("c")
    row0 = wid * _ROWS_PER_W

    pltpu.sync_copy(true_hbm.at[pl.ds(row0, _ROWS_PER_W)], true_v)

    lane = lax.iota(jnp.int32, _L)

    def tile_body(t, acc):
        pltpu.async_copy(
            logits_hbm.at[pl.ds(row0 + t * _TILE_ROWS, _TILE_ROWS)],
            buf,
            dma_sem,
        ).wait()
        for g in range(_GROUPS):
            row_idx = lane + g * _L

            def col_body(_, carry):
                maxv, maxc, base = carry
                maxv, maxc = list(maxv), list(maxc)
                for u in range(_UNROLL):
                    col = base + u
                    v = plsc.load_gather(buf, [row_idx, col])
                    upd = v > maxv[u]
                    maxv[u] = jnp.where(upd, v, maxv[u])
                    maxc[u] = jnp.where(upd, col, maxc[u])
                return tuple(maxv), tuple(maxc), base + _UNROLL

            zero = jnp.zeros((_L,), jnp.int32)
            init = (
                tuple(jnp.full((_L,), -jnp.inf, jnp.float32)
                      for _ in range(_UNROLL)),
                tuple(zero + u for u in range(_UNROLL)),
                zero,
            )
            maxv, maxc, _ = lax.fori_loop(0, _STEPS, col_body, init)

            av, ac = maxv[0], maxc[0]
            for u in range(1, _UNROLL):
                better = (maxv[u] > av) | ((maxv[u] == av) & (maxc[u] < ac))
                av = jnp.where(better, maxv[u], av)
                ac = jnp.where(better, maxc[u], ac)

            true_vec = true_v[pl.ds(t * _TILE_ROWS + g * _L, _L)]
            acc = acc + (ac == true_vec).astype(jnp.int32)
        return acc

    acc = lax.fori_loop(0, _TILES, tile_body, jnp.zeros((_L,), jnp.int32))
    acc_v[...] = acc
    pltpu.sync_copy(acc_v, out_hbm.at[wid])


_recall_counts = functools.partial(
    pl.kernel,
    out_type=jax.ShapeDtypeStruct((_NW, _L), jnp.int32),
    mesh=plsc.VectorSubcoreMesh(
        core_axis_name="c", subcore_axis_name="s",
        num_cores=_NC, num_subcores=_NS,
    ),
    scratch_types=[
        pltpu.VMEM((_TILE_ROWS, _NUM_CLASSES), jnp.float32),
        pltpu.VMEM((_ROWS_PER_W,), jnp.int32),
        pltpu.VMEM((_L,), jnp.int32),
        pltpu.SemaphoreType.DMA,
    ],
    compiler_params=pltpu.CompilerParams(needs_layout_passes=False),
)(_tec_body)


def kernel(true, logits):
    counts = _recall_counts(true, logits)
    return counts.astype(jnp.float32).sum() / jnp.float32(_BATCH)


# speedup vs baseline: 2.3909x; 2.3909x over previous
import functools

import jax
import jax.numpy as jnp
from jax import lax
from jax.experimental import pallas as pl
from jax.experimental.pallas import tpu as pltpu
from jax.experimental.pallas import tpu_sc as plsc

_NUM_CLASSES = 1000
_BATCH = 16384
_NC = 2
_NS = 16
_L = 16
_NW = _NC * _NS
_ROWS_PER_W = _BATCH // _NW
_TILE_ROWS = 32
_TILES = _ROWS_PER_W // _TILE_ROWS
_GROUPS = _TILE_ROWS // _L
_UNROLL = 8
_MAIN_STEPS = 984 // _UNROLL
_HEAD = _L - 1
_TAIL_START = 984


def _tec_body(true_hbm, logits_hbm, out_hbm,
              buf0, buf1, true_v, acc_v, sem0, sem1):
    wid = lax.axis_index("s") * _NC + lax.axis_index("c")
    row0 = wid * _ROWS_PER_W

    pltpu.sync_copy(true_hbm.at[pl.ds(row0, _ROWS_PER_W)], true_v)

    lane = lax.iota(jnp.int32, _L)

    def slab_src(t):
        return logits_hbm.at[pl.ds(row0 + t * _TILE_ROWS, _TILE_ROWS)]

    def process(buf, t, acc):
        for g in range(_GROUPS):
            row_idx = lane + g * _L

            def step(mv, mc, colv, mask=None):
                v = plsc.load_gather(buf, [row_idx, colv])
                upd = v > mv
                if mask is not None:
                    upd = upd & mask
                return jnp.where(upd, v, mv), jnp.where(upd, colv, mc)

            maxv = [jnp.full((_L,), -jnp.inf, jnp.float32)
                    for _ in range(_UNROLL)]
            maxc = [jnp.zeros((_L,), jnp.int32) + u for u in range(_UNROLL)]

            for s in range(_HEAD):
                u = s % _UNROLL
                maxv[u], maxc[u] = step(
                    maxv[u], maxc[u], jnp.zeros((_L,), jnp.int32) + s,
                    mask=lane > s)

            def col_body(_, carry):
                mv, mc, base = carry
                mv, mc = list(mv), list(mc)
                for u in range(_UNROLL):
                    colv = base + u
                    mv[u], mc[u] = step(mv[u], mc[u], colv)
                return tuple(mv), tuple(mc), base + _UNROLL

            out = lax.fori_loop(0, _MAIN_STEPS, col_body,
                                (tuple(maxv), tuple(maxc), lane))
            maxv, maxc = list(out[0]), list(out[1])

            for s in range(_TAIL_START, _NUM_CLASSES):
                u = s % _UNROLL
                maxv[u], maxc[u] = step(
                    maxv[u], maxc[u], lane + s,
                    mask=(lane + s) <= (_NUM_CLASSES - 1))

            av, ac = maxv[0], maxc[0]
            for u in range(1, _UNROLL):
                better = (maxv[u] > av) | ((maxv[u] == av) & (maxc[u] < ac))
                av = jnp.where(better, maxv[u], av)
                ac = jnp.where(better, maxc[u], ac)

            true_vec = true_v[pl.ds(t * _TILE_ROWS + g * _L, _L)]
            acc = acc + (ac == true_vec).astype(jnp.int32)
        return acc

    pltpu.async_copy(slab_src(0), buf0, sem0)

    def pair_body(i, acc):
        for b in range(2):
            buf, sem = (buf0, sem0) if b == 0 else (buf1, sem1)
            nbuf, nsem = (buf1, sem1) if b == 0 else (buf0, sem0)
            t = 2 * i + b

            @pl.when(t + 1 < _TILES)
            def _():
                pltpu.async_copy(slab_src(t + 1), nbuf, nsem)

            pltpu.make_async_copy(slab_src(0), buf, sem).wait()
            acc = process(buf, t, acc)
        return acc

    acc = lax.fori_loop(0, _TILES // 2, pair_body,
                        jnp.zeros((_L,), jnp.int32))
    acc_v[...] = acc
    pltpu.sync_copy(acc_v, out_hbm.at[wid])


_recall_counts = functools.partial(
    pl.kernel,
    out_type=jax.ShapeDtypeStruct((_NW, _L), jnp.int32),
    mesh=plsc.VectorSubcoreMesh(
        core_axis_name="c", subcore_axis_name="s",
        num_cores=_NC, num_subcores=_NS,
    ),
    scratch_types=[
        pltpu.VMEM((_TILE_ROWS, _NUM_CLASSES), jnp.float32),
        pltpu.VMEM((_TILE_ROWS, _NUM_CLASSES), jnp.float32),
        pltpu.VMEM((_ROWS_PER_W,), jnp.int32),
        pltpu.VMEM((_L,), jnp.int32),
        pltpu.SemaphoreType.DMA,
        pltpu.SemaphoreType.DMA,
    ],
    compiler_params=pltpu.CompilerParams(needs_layout_passes=False),
)(_tec_body)


def kernel(true, logits):
    counts = _recall_counts(true, logits)
    return counts.astype(jnp.float32).sum() / jnp.float32(_BATCH)
